# Initial kernel scaffold; baseline (speedup 1.0000x reference)
import jax
import jax.numpy as jnp
from jax.experimental import pallas as pl


def kernel(x, edge_index, edge_attr, W1, b1, W2, b2, LW1, Lb1, LW2, Lb2):
    raise NotImplementedError("write your pallas kernel here")


# speedup vs baseline: 7.6114x; 7.6114x over previous
import functools

import jax
import jax.numpy as jnp
from jax import lax
from jax.experimental import pallas as pl
from jax.experimental.pallas import tpu as pltpu
from jax.experimental.pallas import tpu_sc as plsc

N_NODES = 10000
NP = 10112
N_EDGES = 320000
NW = 32
CHUNK = 128
NCHUNK = 80
EPW = CHUNK * NCHUNK
EP = NW * EPW
H = 64
STRIPE = NP // 16
DUMMY_ROW = N_NODES
DUMMY_COL = N_NODES + 8

_MESH = plsc.VectorSubcoreMesh(core_axis_name="c", subcore_axis_name="s")
_SC_PARAMS = pltpu.CompilerParams(use_tc_tiling_on_sc=False)


def _hist_body(cidx_hbm, init_hbm, ones_hbm, hist_out,
               cidx_v, ones_v, acc, sem):
    c = lax.axis_index("c")
    s = lax.axis_index("s")
    wid = c * 16 + s
    pltpu.sync_copy(init_hbm.at[pl.ds(s * STRIPE, STRIPE)],
                    acc.at[pl.ds(s * STRIPE, STRIPE)])
    pltpu.sync_copy(ones_hbm, ones_v)
    pltpu.sync_copy(cidx_hbm.at[wid], cidx_v)
    plsc.subcore_barrier()

    def step(j, _):
        pltpu.sync_copy(ones_v, acc.at[cidx_v.at[j]], add=True)
        return ()

    lax.fori_loop(0, NCHUNK, step, ())
    plsc.subcore_barrier()
    pltpu.sync_copy(acc.at[pl.ds(s * STRIPE, STRIPE)],
                    hist_out.at[c, pl.ds(s * STRIPE, STRIPE)])


def _sc_hist(cidx, init16, ones16):
    k = pl.kernel(
        _hist_body,
        out_type=jax.ShapeDtypeStruct((2, NP, 16), jnp.float32),
        mesh=_MESH,
        compiler_params=_SC_PARAMS,
        scratch_types=[
            pltpu.VMEM((NCHUNK, CHUNK), jnp.int32),
            pltpu.VMEM((CHUNK, 16), jnp.float32),
            pltpu.VMEM_SHARED((NP, 16), jnp.float32),
            pltpu.SemaphoreType.DMA,
        ],
    )
    return k(cidx, init16, ones16)


def _scatter_body(y_hbm, ridx_hbm, cidx_hbm, init_hbm, s_out,
                  ridx_v, cidx_v, rows_v, acc, sem):
    c = lax.axis_index("c")
    s = lax.axis_index("s")
    wid = c * 16 + s
    pltpu.sync_copy(init_hbm.at[pl.ds(s * STRIPE, STRIPE)],
                    acc.at[pl.ds(s * STRIPE, STRIPE)])
    pltpu.sync_copy(ridx_hbm.at[wid], ridx_v)
    pltpu.sync_copy(cidx_hbm.at[wid], cidx_v)
    plsc.subcore_barrier()

    def step(j, _):
        pltpu.async_copy(y_hbm.at[ridx_v.at[j]], rows_v, sem).wait()
        pltpu.sync_copy(rows_v, acc.at[cidx_v.at[j]], add=True)
        return ()

    lax.fori_loop(0, NCHUNK, step, ())
    plsc.subcore_barrier()
    pltpu.sync_copy(acc.at[pl.ds(s * STRIPE, STRIPE)],
                    s_out.at[c, pl.ds(s * STRIPE, STRIPE)])


def _sc_scatter(y, ridx, cidx, init64):
    k = pl.kernel(
        _scatter_body,
        out_type=jax.ShapeDtypeStruct((2, NP, H), jnp.float32),
        mesh=_MESH,
        compiler_params=_SC_PARAMS,
        scratch_types=[
            pltpu.VMEM((NCHUNK, CHUNK), jnp.int32),
            pltpu.VMEM((NCHUNK, CHUNK), jnp.int32),
            pltpu.VMEM((CHUNK, H), jnp.float32),
            pltpu.VMEM_SHARED((NP, H), jnp.float32),
            pltpu.SemaphoreType.DMA,
        ],
    )
    return k(y, ridx, cidx, init64)


def _edge_body(a_hbm, b_hbm, ridx_hbm, cidx_hbm, s_out,
               ridx_v, cidx_v, ra_v, rb_v, sem_a, sem_b):
    c = lax.axis_index("c")
    s = lax.axis_index("s")
    wid = c * 16 + s
    pltpu.sync_copy(ridx_hbm.at[wid], ridx_v)
    pltpu.sync_copy(cidx_hbm.at[wid], cidx_v)

    def step(j, _):
        da = pltpu.async_copy(a_hbm.at[ridx_v.at[j]], ra_v, sem_a)
        db = pltpu.async_copy(b_hbm.at[cidx_v.at[j]], rb_v, sem_b)
        da.wait()
        db.wait()

        def add_row(r, _):
            for q in range(H // 16):
                sl = pl.ds(q * 16, 16)
                ra_v[r, sl] = ra_v[r, sl] + rb_v[r, sl]
            return ()

        lax.fori_loop(0, CHUNK, add_row, ())
        pltpu.sync_copy(ra_v, s_out.at[pl.ds(wid * EPW + j * CHUNK, CHUNK)])
        return ()

    lax.fori_loop(0, NCHUNK, step, ())


def _sc_edge(a, b, ridx, cidx):
    k = pl.kernel(
        _edge_body,
        out_type=jax.ShapeDtypeStruct((EP, H), jnp.float32),
        mesh=_MESH,
        compiler_params=_SC_PARAMS,
        scratch_types=[
            pltpu.VMEM((NCHUNK, CHUNK), jnp.int32),
            pltpu.VMEM((NCHUNK, CHUNK), jnp.int32),
            pltpu.VMEM((CHUNK, H), jnp.float32),
            pltpu.VMEM((CHUNK, H), jnp.float32),
            pltpu.SemaphoreType.DMA,
            pltpu.SemaphoreType.DMA,
        ],
    )
    return k(a, b, ridx, cidx)


def _t1_body(x_ref, w1_ref, ha_ref, hb_ref, y_ref):
    deg = ha_ref[:, 0:1] + hb_ref[:, 0:1] + 1.0
    dinv = lax.rsqrt(deg)
    xw = jnp.dot(x_ref[...], w1_ref[...], preferred_element_type=jnp.float32)
    y_ref[...] = dinv * xw


def _t2_body(sa_ref, sb_ref, y_ref, ha_ref, hb_ref, w2_ref, b1_ref, y2_ref):
    deg = ha_ref[:, 0:1] + hb_ref[:, 0:1] + 1.0
    dinv = lax.rsqrt(deg)
    h1 = jnp.maximum(
        dinv * (sa_ref[...] + sb_ref[...] + y_ref[...]) + b1_ref[...], 0.0)
    y2_ref[...] = dinv * jnp.dot(h1, w2_ref[...],
                                 preferred_element_type=jnp.float32)


def _t3_body(sa_ref, sb_ref, y_ref, ha_ref, hb_ref, b2_ref,
             lw1a_ref, lw1b_ref, lb1_ref, a_ref, b_ref):
    deg = ha_ref[:, 0:1] + hb_ref[:, 0:1] + 1.0
    dinv = lax.rsqrt(deg)
    h2 = jnp.maximum(
        dinv * (sa_ref[...] + sb_ref[...] + y_ref[...]) + b2_ref[...], 0.0)
    a_ref[...] = jnp.dot(h2, lw1a_ref[...],
                         preferred_element_type=jnp.float32) + lb1_ref[...]
    b_ref[...] = jnp.dot(h2, lw1b_ref[...],
                         preferred_element_type=jnp.float32)


def _t4_body(s_ref, ea_ref, c_ref, lw2_ref, lb2_ref, o_ref):
    z = jnp.maximum(s_ref[...] + ea_ref[...] * c_ref[...], 0.0)
    o_ref[...] = jnp.sum(z * lw2_ref[...], axis=1) + lb2_ref[0, 0]


_EB = 8192


def _tc_final(s_edges, ea, c_row, lw2_row, lb2):
    return pl.pallas_call(
        _t4_body,
        grid=(EP // _EB,),
        in_specs=[
            pl.BlockSpec((_EB, H), lambda i: (i, 0)),
            pl.BlockSpec((_EB, 1), lambda i: (i, 0)),
            pl.BlockSpec((1, H), lambda i: (0, 0)),
            pl.BlockSpec((1, H), lambda i: (0, 0)),
            pl.BlockSpec((1, 1), lambda i: (0, 0)),
        ],
        out_specs=pl.BlockSpec((_EB,), lambda i: (i,)),
        out_shape=jax.ShapeDtypeStruct((EP,), jnp.float32),
    )(s_edges, ea, c_row, lw2_row, lb2)


@jax.jit
def kernel(x, edge_index, edge_attr, W1, b1, W2, b2, LW1, Lb1, LW2, Lb2):
    ei = edge_index.astype(jnp.int32)
    row = jnp.concatenate(
        [ei[0], jnp.full((EP - N_EDGES,), DUMMY_ROW, jnp.int32)])
    col = jnp.concatenate(
        [ei[1], jnp.full((EP - N_EDGES,), DUMMY_COL, jnp.int32)])
    ridx = row.reshape(NW, NCHUNK, CHUNK)
    cidx = col.reshape(NW, NCHUNK, CHUNK)

    xp = jnp.zeros((NP, x.shape[1]), jnp.float32).at[:N_NODES].set(x)
    init16 = jnp.zeros((NP, 16), jnp.float32)
    init64 = jnp.zeros((NP, H), jnp.float32)
    ones16 = jnp.zeros((CHUNK, 16), jnp.float32).at[:, 0].set(1.0)

    hist = _sc_hist(cidx, init16, ones16)
    ha, hb = hist[0], hist[1]

    y1 = pl.pallas_call(
        _t1_body,
        out_shape=jax.ShapeDtypeStruct((NP, H), jnp.float32),
    )(xp, W1, ha, hb)

    s1 = _sc_scatter(y1, ridx, cidx, init64)

    y2 = pl.pallas_call(
        _t2_body,
        out_shape=jax.ShapeDtypeStruct((NP, H), jnp.float32),
    )(s1[0], s1[1], y1, ha, hb, W2, b1.reshape(1, H))

    s2 = _sc_scatter(y2, ridx, cidx, init64)

    a_nodes, b_nodes = pl.pallas_call(
        _t3_body,
        out_shape=[
            jax.ShapeDtypeStruct((NP, H), jnp.float32),
            jax.ShapeDtypeStruct((NP, H), jnp.float32),
        ],
    )(s2[0], s2[1], y2, ha, hb, b2.reshape(1, H),
      LW1[:H], LW1[H:2 * H], Lb1.reshape(1, H))

    s_edges = _sc_edge(a_nodes, b_nodes, ridx, cidx)

    ea = jnp.zeros((EP, 1), jnp.float32).at[:N_EDGES].set(edge_attr)
    out = _tc_final(s_edges, ea, LW1[2 * H:2 * H + 1],
                    LW2.reshape(1, H), Lb2.reshape(1, 1))
    return out[:N_EDGES]
